# baseline (speedup 1.0000x reference)
import functools

import jax
import jax.numpy as jnp
from jax import lax
from jax.experimental import pallas as pl
from jax.experimental.pallas import tpu as pltpu
from jax.experimental.pallas import tpu_sc as plsc

VOCAB = 1_000_000
D = 64
SEQ = 1024
BATCH = 1024
FLAT = BATCH * SEQ

NUM_CORES = 2
NUM_SUBCORES = 16
NW = NUM_CORES * NUM_SUBCORES
PER_W = FLAT // NW
C = 128
NCHUNK = PER_W // C
LANES = 16

_mesh = plsc.VectorSubcoreMesh(core_axis_name="c", subcore_axis_name="s")


@functools.partial(
    pl.kernel,
    mesh=_mesh,
    compiler_params=pltpu.CompilerParams(use_tc_tiling_on_sc=False),
    out_type=jax.ShapeDtypeStruct((FLAT, D), jnp.float32),
    scratch_types=[
        pltpu.VMEM((SEQ, D), jnp.float32),
        pltpu.VMEM((C,), jnp.int32),
        pltpu.VMEM((C, D), jnp.float32),
        pltpu.SemaphoreType.DMA,
    ],
)
def _embed(ids_hbm, tok_hbm, pos_hbm, out_hbm, pos_v, idx_v, rows_v, gsem):
    wid = lax.axis_index("s") * NUM_CORES + lax.axis_index("c")
    base = wid * PER_W
    pltpu.sync_copy(pos_hbm, pos_v)

    def chunk_body(g, carry):
        start = base + g * C
        pltpu.sync_copy(ids_hbm.at[pl.ds(start, C)], idx_v)
        pltpu.async_copy(tok_hbm.at[idx_v], rows_v, gsem).wait()
        p0 = (g * C) % SEQ

        def add_row(i, carry2):
            for j in range(D // LANES):
                sl = pl.ds(j * LANES, LANES)
                rows_v[i, sl] = rows_v[i, sl] + pos_v[p0 + i, sl]
            return carry2

        lax.fori_loop(0, C, add_row, 0, unroll=2)
        pltpu.sync_copy(rows_v, out_hbm.at[pl.ds(start, C)])
        return carry

    lax.fori_loop(0, NCHUNK, chunk_body, 0)


def kernel(input_ids, token_table, pos_table):
    b, s = input_ids.shape
    ids_flat = input_ids.reshape(FLAT).astype(jnp.int32)
    out = _embed(ids_flat, token_table, pos_table)
    return out.reshape(b, s, D)


# speedup vs baseline: 1.1984x; 1.1984x over previous
import functools

import jax
import jax.numpy as jnp
from jax import lax
from jax.experimental import pallas as pl
from jax.experimental.pallas import tpu as pltpu
from jax.experimental.pallas import tpu_sc as plsc

VOCAB = 1_000_000
D = 64
SEQ = 1024
BATCH = 1024
FLAT = BATCH * SEQ

NUM_CORES = 2
NUM_SUBCORES = 16
NW = NUM_CORES * NUM_SUBCORES
PER_W = FLAT // NW
C = 128
NCHUNK = PER_W // C
N2 = NCHUNK // 2
LANES = 16

_mesh = plsc.VectorSubcoreMesh(core_axis_name="c", subcore_axis_name="s")


@functools.partial(
    pl.kernel,
    mesh=_mesh,
    compiler_params=pltpu.CompilerParams(use_tc_tiling_on_sc=False),
    out_type=jax.ShapeDtypeStruct((FLAT, D), jnp.float32),
    scratch_types=[
        pltpu.VMEM((SEQ, D), jnp.float32),
        pltpu.VMEM((2, C), jnp.int32),
        pltpu.VMEM((2, C, D), jnp.float32),
        pltpu.SemaphoreType.DMA((2,)),
        pltpu.SemaphoreType.DMA((2,)),
        pltpu.SemaphoreType.DMA((2,)),
    ],
)
def _embed(ids_hbm, tok_hbm, pos_hbm, out_hbm, pos_v, idx_v, rows_v,
           gsem, osem, isem):
    wid = lax.axis_index("s") * NUM_CORES + lax.axis_index("c")
    base = wid * PER_W
    pltpu.sync_copy(pos_hbm, pos_v)

    pltpu.sync_copy(ids_hbm.at[pl.ds(base, C)], idx_v.at[0])
    pltpu.async_copy(ids_hbm.at[pl.ds(base + C, C)], idx_v.at[1], isem.at[1])
    pltpu.async_copy(tok_hbm.at[idx_v.at[0]], rows_v.at[0], gsem.at[0])

    def pair_body(g2, carry):
        for b in (0, 1):
            nb = 1 - b
            g = g2 * 2 + b
            start = base + g * C

            @pl.when((g2 > 0) | (b == 1))
            def _wait_prev_store():
                pltpu.make_async_copy(
                    rows_v.at[nb], out_hbm.at[pl.ds(base, C)], osem.at[nb]
                ).wait()

            @pl.when(g + 1 < NCHUNK)
            def _launch_next_gather():
                pltpu.make_async_copy(
                    ids_hbm.at[pl.ds(base, C)], idx_v.at[nb], isem.at[nb]
                ).wait()
                pltpu.async_copy(
                    tok_hbm.at[idx_v.at[nb]], rows_v.at[nb], gsem.at[nb])

            pltpu.make_async_copy(
                tok_hbm.at[idx_v.at[b]], rows_v.at[b], gsem.at[b]
            ).wait()

            @pl.when(g + 2 < NCHUNK)
            def _prefetch_idx():
                pltpu.async_copy(
                    ids_hbm.at[pl.ds(start + 2 * C, C)], idx_v.at[b],
                    isem.at[b])

            p0 = (g * C) % SEQ

            def add_row(i, carry2):
                for j in range(D // LANES):
                    sl = pl.ds(j * LANES, LANES)
                    rows_v[b, i, sl] = rows_v[b, i, sl] + pos_v[p0 + i, sl]
                return carry2

            lax.fori_loop(0, C, add_row, 0, unroll=2)

            pltpu.async_copy(
                rows_v.at[b], out_hbm.at[pl.ds(start, C)], osem.at[b])
        return carry

    lax.fori_loop(0, N2, pair_body, 0)
    pltpu.make_async_copy(
        rows_v.at[1], out_hbm.at[pl.ds(base, C)], osem.at[1]).wait()


def kernel(input_ids, token_table, pos_table):
    b, s = input_ids.shape
    ids_flat = input_ids.reshape(FLAT).astype(jnp.int32)
    out = _embed(ids_flat, token_table, pos_table)
    return out.reshape(b, s, D)


# speedup vs baseline: 1.5768x; 1.3157x over previous
import functools

import jax
import jax.numpy as jnp
from jax import lax
from jax.experimental import pallas as pl
from jax.experimental.pallas import tpu as pltpu
from jax.experimental.pallas import tpu_sc as plsc

VOCAB = 1_000_000
D = 64
SEQ = 1024
BATCH = 1024
FLAT = BATCH * SEQ

NUM_CORES = 2
NUM_SUBCORES = 16
NW = NUM_CORES * NUM_SUBCORES
PER_W = FLAT // NW
C = 128
NCHUNK = PER_W // C
N2 = NCHUNK // 2
LANES = 16

_mesh = plsc.VectorSubcoreMesh(core_axis_name="c", subcore_axis_name="s")


@functools.partial(
    pl.kernel,
    mesh=_mesh,
    compiler_params=pltpu.CompilerParams(use_tc_tiling_on_sc=False),
    out_type=jax.ShapeDtypeStruct((FLAT, D), jnp.float32),
    scratch_types=[
        pltpu.VMEM((SEQ, D), jnp.float32),
        pltpu.VMEM((2, C), jnp.int32),
        pltpu.VMEM((2, C, D), jnp.float32),
        pltpu.SemaphoreType.DMA((2,)),
        pltpu.SemaphoreType.DMA((2,)),
        pltpu.SemaphoreType.DMA((2,)),
    ],
)
def _embed(ids_hbm, tok_hbm, pos_hbm, out_hbm, pos_v, idx_v, rows_v,
           gsem, osem, isem):
    wid = lax.axis_index("s") * NUM_CORES + lax.axis_index("c")
    base = wid * PER_W
    pltpu.sync_copy(pos_hbm, pos_v)

    def prefill(buf, p0):
        @plsc.parallel_loop(0, C, unroll=4)
        def _fill(i):
            for j in range(D // LANES):
                sl = pl.ds(j * LANES, LANES)
                rows_v[buf, i, sl] = pos_v[p0 + i, sl]

    prefill(0, 0)
    pltpu.sync_copy(ids_hbm.at[pl.ds(base, C)], idx_v.at[0])
    pltpu.async_copy(ids_hbm.at[pl.ds(base + C, C)], idx_v.at[1], isem.at[1])
    pltpu.async_copy(tok_hbm.at[idx_v.at[0]], rows_v.at[0], gsem.at[0],
                     add=True)

    def pair_body(g2, carry):
        for b in (0, 1):
            nb = 1 - b
            g = g2 * 2 + b
            start = base + g * C

            @pl.when((g2 > 0) | (b == 1))
            def _wait_prev_store():
                pltpu.make_async_copy(
                    rows_v.at[nb], out_hbm.at[pl.ds(base, C)], osem.at[nb]
                ).wait()

            @pl.when(g + 1 < NCHUNK)
            def _launch_next_gather():
                prefill(nb, ((g + 1) * C) % SEQ)
                pltpu.make_async_copy(
                    ids_hbm.at[pl.ds(base, C)], idx_v.at[nb], isem.at[nb]
                ).wait()
                pltpu.async_copy(
                    tok_hbm.at[idx_v.at[nb]], rows_v.at[nb], gsem.at[nb],
                    add=True)

            pltpu.make_async_copy(
                tok_hbm.at[idx_v.at[b]], rows_v.at[b], gsem.at[b]
            ).wait()

            @pl.when(g + 2 < NCHUNK)
            def _prefetch_idx():
                pltpu.async_copy(
                    ids_hbm.at[pl.ds(start + 2 * C, C)], idx_v.at[b],
                    isem.at[b])

            pltpu.async_copy(
                rows_v.at[b], out_hbm.at[pl.ds(start, C)], osem.at[b])
        return carry

    lax.fori_loop(0, N2, pair_body, 0)
    pltpu.make_async_copy(
        rows_v.at[1], out_hbm.at[pl.ds(base, C)], osem.at[1]).wait()


def kernel(input_ids, token_table, pos_table):
    b, s = input_ids.shape
    ids_flat = input_ids.reshape(FLAT).astype(jnp.int32)
    out = _embed(ids_flat, token_table, pos_table)
    return out.reshape(b, s, D)


# speedup vs baseline: 1.6475x; 1.0449x over previous
import functools

import jax
import jax.numpy as jnp
from jax import lax
from jax.experimental import pallas as pl
from jax.experimental.pallas import tpu as pltpu
from jax.experimental.pallas import tpu_sc as plsc

VOCAB = 1_000_000
D = 64
SEQ = 1024
BATCH = 1024
FLAT = BATCH * SEQ

NUM_CORES = 2
NUM_SUBCORES = 16
NW = NUM_CORES * NUM_SUBCORES
PER_W = FLAT // NW
C = 128
NCHUNK = PER_W // C
NBUF = 4
N4 = NCHUNK // NBUF
LANES = 16

_mesh = plsc.VectorSubcoreMesh(core_axis_name="c", subcore_axis_name="s")


@functools.partial(
    pl.kernel,
    mesh=_mesh,
    compiler_params=pltpu.CompilerParams(use_tc_tiling_on_sc=False),
    out_type=jax.ShapeDtypeStruct((FLAT, D), jnp.float32),
    scratch_types=[
        pltpu.VMEM((SEQ, D), jnp.float32),
        pltpu.VMEM((NBUF, C), jnp.int32),
        pltpu.VMEM((NBUF, C, D), jnp.float32),
        pltpu.SemaphoreType.DMA((NBUF,)),
        pltpu.SemaphoreType.DMA((NBUF,)),
        pltpu.SemaphoreType.DMA((NBUF,)),
    ],
)
def _embed(ids_hbm, tok_hbm, pos_hbm, out_hbm, pos_v, idx_v, rows_v,
           gsem, osem, isem):
    wid = lax.axis_index("s") * NUM_CORES + lax.axis_index("c")
    base = wid * PER_W
    pltpu.sync_copy(pos_hbm, pos_v)

    def prefill(buf, p0):
        @plsc.parallel_loop(0, C, unroll=4)
        def _fill(i):
            for j in range(D // LANES):
                sl = pl.ds(j * LANES, LANES)
                rows_v[buf, i, sl] = pos_v[p0 + i, sl]

    def wait_store(buf):
        pltpu.make_async_copy(
            rows_v.at[buf], out_hbm.at[pl.ds(base, C)], osem.at[buf]).wait()

    def wait_gather(buf):
        pltpu.make_async_copy(
            tok_hbm.at[idx_v.at[buf]], rows_v.at[buf], gsem.at[buf]).wait()

    for k in range(NBUF):
        pltpu.async_copy(ids_hbm.at[pl.ds(base + k * C, C)], idx_v.at[k],
                         isem.at[k])

    def quad_body(o, carry):
        for b in range(NBUF):
            g = o * NBUF + b
            bl = (b + NBUF - 2) % NBUF

            @pl.when(o > 0)
            def _wait_prev_store():
                wait_store(b)

            prefill(b, (g * C) % SEQ)
            pltpu.make_async_copy(
                ids_hbm.at[pl.ds(base, C)], idx_v.at[b], isem.at[b]).wait()
            pltpu.async_copy(tok_hbm.at[idx_v.at[b]], rows_v.at[b],
                             gsem.at[b], add=True)

            @pl.when((o > 0) | (b >= 2))
            def _retire():
                wait_gather(bl)
                pltpu.async_copy(
                    rows_v.at[bl],
                    out_hbm.at[pl.ds(base + (g - 2) * C, C)], osem.at[bl])

                @pl.when(g + 2 < NCHUNK)
                def _prefetch_idx():
                    pltpu.async_copy(
                        ids_hbm.at[pl.ds(base + (g + 2) * C, C)],
                        idx_v.at[bl], isem.at[bl])
        return carry

    lax.fori_loop(0, N4, quad_body, 0)

    for g in (NCHUNK - 2, NCHUNK - 1):
        b = g % NBUF
        wait_gather(b)
        pltpu.async_copy(rows_v.at[b], out_hbm.at[pl.ds(base + g * C, C)],
                         osem.at[b])
    for b in range(NBUF):
        wait_store(b)


def kernel(input_ids, token_table, pos_table):
    b, s = input_ids.shape
    ids_flat = input_ids.reshape(FLAT).astype(jnp.int32)
    out = _embed(ids_flat, token_table, pos_table)
    return out.reshape(b, s, D)
